# baseline (speedup 1.0000x reference)
import jax
import jax.numpy as jnp
from jax.experimental import pallas as pl
from jax.experimental.pallas import tpu as pltpu

_BM = 256
_NBUF = 4


def _matmul_prelu_kernel(w_ref, adj_hbm, seq_ref, out_ref, bufs, sems):
    n = adj_hbm.shape[0]
    nchunk = n // _BM

    def copy_in(chunk):
        buf = chunk % _NBUF
        pltpu.make_async_copy(
            adj_hbm.at[pl.ds(chunk * _BM, _BM), :],
            bufs.at[buf],
            sems.at[buf],
        ).start()

    for j in range(min(_NBUF, nchunk)):
        copy_in(j)

    w = w_ref[0, 0]
    for i in range(nchunk):
        buf = i % _NBUF
        pltpu.make_async_copy(
            adj_hbm.at[pl.ds(i * _BM, _BM), :], bufs.at[buf], sems.at[buf]
        ).wait()
        y = jnp.dot(bufs[buf], seq_ref[...], preferred_element_type=jnp.float32)
        out_ref[pl.ds(i * _BM, _BM), :] = jnp.where(y >= 0, y, w * y)
        nxt = i + _NBUF
        if nxt < nchunk:
            copy_in(nxt)


def kernel(seq, adj_avg, prelu_weight):
    n, d = seq.shape
    w2d = prelu_weight.reshape(1, 1)
    return pl.pallas_call(
        _matmul_prelu_kernel,
        in_specs=[
            pl.BlockSpec(memory_space=pltpu.SMEM),
            pl.BlockSpec(memory_space=pltpu.MemorySpace.HBM),
            pl.BlockSpec(memory_space=pltpu.VMEM),
        ],
        out_specs=pl.BlockSpec(memory_space=pltpu.VMEM),
        out_shape=jax.ShapeDtypeStruct((n, d), jnp.float32),
        scratch_shapes=[
            pltpu.VMEM((_NBUF, _BM, n), jnp.float32),
            pltpu.SemaphoreType.DMA((_NBUF,)),
        ],
    )(w2d, adj_avg, seq)
